# baseline (speedup 1.0000x reference)
import functools

import jax
import jax.numpy as jnp
from jax import lax
from jax.experimental import pallas as pl
from jax.experimental.pallas import tpu as pltpu
from jax.experimental.pallas import tpu_sc as plsc

B = 4096
D = 16
L = 16
NC = 2
NS = 16
NW = NC * NS
RPW = B // NW
R = 2
NCHUNK = B // L


def _koleo_sc(xt_hbm, x_hbm, pmin_hbm, xt_v, xrows_v, sqj_v, pmin_v):
    cid = lax.axis_index("c")
    sid = lax.axis_index("s")
    wid = sid * NC + cid
    row0 = wid * RPW

    pltpu.sync_copy(xt_hbm, xt_v)
    pltpu.sync_copy(x_hbm.at[pl.ds(row0 * D, RPW * D)], xrows_v)

    def sq_chunk(c, carry):
        base = c * L
        ps = []
        for p0 in range(0, D, 4):
            s = None
            for d in range(p0, p0 + 4):
                v = xt_v[pl.ds(d * B + base, L)]
                s = v * v if s is None else s + v * v
            ps.append(s)
        sqj_v[pl.ds(base, L)] = (ps[0] + ps[1]) + (ps[2] + ps[3])
        return carry

    lax.fori_loop(0, NCHUNK, sq_chunk, 0)

    inf = jnp.full((L,), jnp.inf, jnp.float32)
    iota = lax.iota(jnp.int32, L)
    dsplat = [jnp.full((L,), d, jnp.int32) for d in range(D)]

    def group(g, carry):
        i0 = row0 + g * R
        bs = []
        for r in range(R):
            xi = xrows_v[pl.ds((g * R + r) * D, D)] * -2.0
            for d in range(D):
                bs.append(xi.at[dsplat[d]].get(mode="promise_in_bounds"))

        def chunk(c, accs):
            base = c * L
            xj = [xt_v[pl.ds(d * B + base, L)] for d in range(D)]
            sj = sqj_v[pl.ds(base, L)]
            jidx = iota + base
            out = []
            for r in range(R):
                ps = []
                for p0 in range(0, D, 4):
                    m = bs[r * D + p0] * xj[p0]
                    for d in range(p0 + 1, p0 + 4):
                        m = m + bs[r * D + d] * xj[d]
                    ps.append(m)
                m = ((ps[0] + ps[1]) + (ps[2] + ps[3])) + sj
                diag = jidx == jnp.full((L,), i0 + r, jnp.int32)
                m = jnp.where(diag, inf, m)
                out.append(jnp.minimum(accs[r], m))
            return tuple(out)

        accs = lax.fori_loop(0, NCHUNK, chunk, (inf,) * R)
        for r in range(R):
            pmin_v[pl.ds((g * R + r) * L, L)] = accs[r]
        return carry

    lax.fori_loop(0, RPW // R, group, 0)
    pltpu.sync_copy(pmin_v, pmin_hbm.at[pl.ds(row0 * L, RPW * L)])


def _sc_partial_min(xt_flat, x_flat):
    mesh = plsc.VectorSubcoreMesh(core_axis_name="c", subcore_axis_name="s")
    kern = pl.kernel(
        _koleo_sc,
        mesh=mesh,
        out_type=jax.ShapeDtypeStruct((B * L,), jnp.float32),
        scratch_types=[
            pltpu.VMEM((D * B,), jnp.float32),
            pltpu.VMEM((RPW * D,), jnp.float32),
            pltpu.VMEM((B,), jnp.float32),
            pltpu.VMEM((RPW * L,), jnp.float32),
        ],
    )
    return kern(xt_flat, x_flat)


def _epilogue_tc(x_ref, pmin_ref, out_ref):
    x = x_ref[...]
    pm = pmin_ref[...]
    sq = jnp.sum(x * x, axis=1)
    d2 = sq + jnp.min(pm, axis=1)
    d2 = jnp.maximum(d2, 0.0)
    nn = jnp.sqrt(d2)
    out_ref[0, 0] = -jnp.mean(jnp.log(nn + 1e-8))


def kernel(student_output):
    x = student_output
    xt_flat = x.T.reshape(-1)
    pmin = _sc_partial_min(xt_flat, x.reshape(-1)).reshape(B, L)
    loss = pl.pallas_call(
        _epilogue_tc,
        out_shape=jax.ShapeDtypeStruct((1, 1), jnp.float32),
        out_specs=pl.BlockSpec(memory_space=pltpu.SMEM),
    )(x, pmin)
    return loss[0, 0]


# speedup vs baseline: 5.6401x; 5.6401x over previous
import functools

import jax
import jax.numpy as jnp
from jax import lax
from jax.experimental import pallas as pl
from jax.experimental.pallas import tpu as pltpu
from jax.experimental.pallas import tpu_sc as plsc

B = 4096
D = 16
L = 16
NC = 2
NS = 16
NW = NC * NS
NCHUNK = B // L
R = 2

SC_ROWS = 256
TC_ROWS = B - SC_ROWS
SC_ROW0 = TC_ROWS
SC_RPW = SC_ROWS // NW
TC_BM = 384
assert TC_ROWS % TC_BM == 0 and SC_RPW % R == 0


def _koleo_sc(xt_hbm, x_hbm, pmin_hbm, xt_v, xrows_v, sqj_v, pmin_v):
    cid = lax.axis_index("c")
    sid = lax.axis_index("s")
    wid = sid * NC + cid
    lrow0 = wid * SC_RPW
    row0 = SC_ROW0 + lrow0

    pltpu.sync_copy(xt_hbm, xt_v)
    pltpu.sync_copy(x_hbm.at[pl.ds(row0 * D, SC_RPW * D)], xrows_v)

    def sq_chunk(c, carry):
        base = c * L
        ps = []
        for p0 in range(0, D, 4):
            s = None
            for d in range(p0, p0 + 4):
                v = xt_v[pl.ds(d * B + base, L)]
                s = v * v if s is None else s + v * v
            ps.append(s)
        sqj_v[pl.ds(base, L)] = (ps[0] + ps[1]) + (ps[2] + ps[3])
        return carry

    lax.fori_loop(0, NCHUNK, sq_chunk, 0)

    inf = jnp.full((L,), jnp.inf, jnp.float32)
    iota = lax.iota(jnp.int32, L)
    dsplat = [jnp.full((L,), d, jnp.int32) for d in range(D)]

    def group(g, carry):
        i0 = row0 + g * R
        bs = []
        for r in range(R):
            xi = xrows_v[pl.ds((g * R + r) * D, D)] * -2.0
            for d in range(D):
                bs.append(xi.at[dsplat[d]].get(mode="promise_in_bounds"))

        def chunk(c, accs):
            base = c * L
            xj = [xt_v[pl.ds(d * B + base, L)] for d in range(D)]
            sj = sqj_v[pl.ds(base, L)]
            jidx = iota + base
            out = []
            for r in range(R):
                ps = []
                for p0 in range(0, D, 4):
                    m = bs[r * D + p0] * xj[p0]
                    for d in range(p0 + 1, p0 + 4):
                        m = m + bs[r * D + d] * xj[d]
                    ps.append(m)
                m = ((ps[0] + ps[1]) + (ps[2] + ps[3])) + sj
                diag = jidx == jnp.full((L,), i0 + r, jnp.int32)
                m = jnp.where(diag, inf, m)
                out.append(jnp.minimum(accs[r], m))
            return tuple(out)

        accs = lax.fori_loop(0, NCHUNK, chunk, (inf,) * R)
        for r in range(R):
            pmin_v[pl.ds((g * R + r) * L, L)] = accs[r]
        return carry

    lax.fori_loop(0, SC_RPW // R, group, 0)
    pltpu.sync_copy(pmin_v, pmin_hbm.at[pl.ds(lrow0 * L, SC_RPW * L)])


def _sc_partial_min(xt_flat, x_flat):
    mesh = plsc.VectorSubcoreMesh(core_axis_name="c", subcore_axis_name="s")
    kern = pl.kernel(
        _koleo_sc,
        mesh=mesh,
        out_type=jax.ShapeDtypeStruct((SC_ROWS * L,), jnp.float32),
        scratch_types=[
            pltpu.VMEM((D * B,), jnp.float32),
            pltpu.VMEM((SC_RPW * D,), jnp.float32),
            pltpu.VMEM((B,), jnp.float32),
            pltpu.VMEM((SC_RPW * L,), jnp.float32),
        ],
    )
    return kern(xt_flat, x_flat)


def _koleo_tc(x_ref, xt_ref, out_ref):
    xt = xt_ref[...]
    sqj = jnp.sum(xt * xt, axis=0)
    jc = lax.broadcasted_iota(jnp.int32, (TC_BM, B), 1)

    def block(b, carry):
        xi = x_ref[pl.ds(b * TC_BM, TC_BM), :]
        scores = jax.lax.dot(xi, xt,
                             preferred_element_type=jnp.float32)
        part = sqj[None, :] - 2.0 * scores
        jr = lax.broadcasted_iota(jnp.int32, (TC_BM, B), 0) + b * TC_BM
        part = jnp.where(jr == jc, jnp.inf, part)
        out_ref[pl.ds(b * TC_BM, TC_BM)] = jnp.min(part, axis=1)
        return carry

    lax.fori_loop(0, TC_ROWS // TC_BM, block, 0)


def _tc_partial_min(x, xt):
    return pl.pallas_call(
        _koleo_tc,
        out_shape=jax.ShapeDtypeStruct((TC_ROWS,), jnp.float32),
    )(x, xt)


def _epilogue_tc(x_ref, tcmin_ref, pmin_ref, out_ref):
    x = x_ref[...]
    sq = jnp.sum(x * x, axis=1)
    tc = tcmin_ref[...]
    pm = pmin_ref[...]
    d2_tc = sq[:TC_ROWS] + tc
    d2_sc = sq[TC_ROWS:] + jnp.min(pm, axis=1)
    d2 = jnp.concatenate([d2_tc, d2_sc])
    d2 = jnp.maximum(d2, 0.0)
    nn = jnp.sqrt(d2)
    out_ref[0, 0] = -jnp.mean(jnp.log(nn + 1e-8))


def kernel(student_output):
    x = student_output
    xt = x.T
    xt_flat = xt.reshape(-1)
    pmin = _sc_partial_min(xt_flat, x.reshape(-1)).reshape(SC_ROWS, L)
    tcmin = _tc_partial_min(x, xt)
    loss = pl.pallas_call(
        _epilogue_tc,
        out_shape=jax.ShapeDtypeStruct((1, 1), jnp.float32),
        out_specs=pl.BlockSpec(memory_space=pltpu.SMEM),
    )(x, tcmin, pmin)
    return loss[0, 0]


# speedup vs baseline: 8.2232x; 1.4580x over previous
import functools

import jax
import jax.numpy as jnp
from jax import lax
from jax.experimental import pallas as pl
from jax.experimental.pallas import tpu as pltpu
from jax.experimental.pallas import tpu_sc as plsc

B = 4096
D = 16
L = 16
NC = 2
NS = 16
NW = NC * NS
NCHUNK = B // L
R = 2

SC_ROWS = 128
TC_ROWS = B - SC_ROWS
SC_ROW0 = TC_ROWS
CW = B // NW
CCH = CW // L
TC_BM = 384
TC_BLOCKS = [(lo, min(lo + TC_BM, TC_ROWS)) for lo in range(0, TC_ROWS, TC_BM)]
assert SC_ROWS % R == 0 and all((hi - lo) % 128 == 0 for lo, hi in TC_BLOCKS)


def _koleo_sc(xtp_hbm, x_hbm, pmin_hbm, xt_v, xrows_v, pmin_v):
    cid = lax.axis_index("c")
    sid = lax.axis_index("s")
    wid = sid * NC + cid
    c0 = wid * CW

    pltpu.sync_copy(xtp_hbm.at[pl.ds(wid * D * CW, D * CW)], xt_v)
    pltpu.sync_copy(x_hbm.at[pl.ds(SC_ROW0 * D, SC_ROWS * D)], xrows_v)

    sjs = []
    for c in range(CCH):
        base = c * L
        ps = []
        for p0 in range(0, D, 4):
            s = None
            for d in range(p0, p0 + 4):
                v = xt_v[pl.ds(d * CW + base, L)]
                s = v * v if s is None else s + v * v
            ps.append(s)
        sjs.append((ps[0] + ps[1]) + (ps[2] + ps[3]))

    inf = jnp.full((L,), jnp.inf, jnp.float32)
    iota = lax.iota(jnp.int32, L)
    dsplat = [jnp.full((L,), d, jnp.int32) for d in range(D)]

    def group(g, carry):
        i0 = SC_ROW0 + g * R
        bs = []
        for r in range(R):
            xi = xrows_v[pl.ds((g * R + r) * D, D)] * -2.0
            for d in range(D):
                bs.append(xi.at[dsplat[d]].get(mode="promise_in_bounds"))
        accs = [inf] * R
        for c in range(CCH):
            base = c * L
            xj = [xt_v[pl.ds(d * CW + base, L)] for d in range(D)]
            jidx = iota + (c0 + base)
            for r in range(R):
                ps = []
                for p0 in range(0, D, 4):
                    m = bs[r * D + p0] * xj[p0]
                    for d in range(p0 + 1, p0 + 4):
                        m = m + bs[r * D + d] * xj[d]
                    ps.append(m)
                m = ((ps[0] + ps[1]) + (ps[2] + ps[3])) + sjs[c]
                diag = jidx == jnp.full((L,), i0 + r, jnp.int32)
                m = jnp.where(diag, inf, m)
                accs[r] = jnp.minimum(accs[r], m)
        for r in range(R):
            pmin_v[pl.ds((g * R + r) * L, L)] = accs[r]
        return carry

    lax.fori_loop(0, SC_ROWS // R, group, 0)
    pltpu.sync_copy(pmin_v, pmin_hbm.at[pl.ds(wid * SC_ROWS * L, SC_ROWS * L)])


def _sc_partial_min(xtp_flat, x_flat):
    mesh = plsc.VectorSubcoreMesh(core_axis_name="c", subcore_axis_name="s")
    kern = pl.kernel(
        _koleo_sc,
        mesh=mesh,
        out_type=jax.ShapeDtypeStruct((NW * SC_ROWS * L,), jnp.float32),
        scratch_types=[
            pltpu.VMEM((D * CW,), jnp.float32),
            pltpu.VMEM((SC_ROWS * D,), jnp.float32),
            pltpu.VMEM((SC_ROWS * L,), jnp.float32),
        ],
    )
    return kern(xtp_flat, x_flat)


def _koleo_tc(x_ref, out_ref, a_ref, b_ref):
    x = x_ref[...]
    sq = jnp.sum(x * x, axis=1, keepdims=True)
    a_ref[...] = jnp.concatenate([x, jnp.ones((B, 1), jnp.float32)], axis=1)
    b_ref[...] = jnp.concatenate([-2.0 * x, sq], axis=1)
    eyes = {}
    for lo, hi in TC_BLOCKS:
        sz = hi - lo
        if sz not in eyes:
            eyes[sz] = (lax.broadcasted_iota(jnp.int32, (sz, sz), 0)
                        == lax.broadcasted_iota(jnp.int32, (sz, sz), 1))
    inf = jnp.float32(jnp.inf)
    bfull = b_ref[...]

    for lo, hi in TC_BLOCKS:
        eye = eyes[hi - lo]
        ai = a_ref[lo:hi, :]
        part = lax.dot_general(ai, bfull,
                               (((1,), (1,)), ((), ())),
                               preferred_element_type=jnp.float32)
        strip = jnp.where(eye, inf, part[:, lo:hi])
        m = jnp.min(strip, axis=1)
        if lo > 0:
            m = jnp.minimum(m, jnp.min(part[:, :lo], axis=1))
        if hi < B:
            m = jnp.minimum(m, jnp.min(part[:, hi:], axis=1))
        out_ref[lo:hi] = m


def _tc_partial_min(x):
    return pl.pallas_call(
        _koleo_tc,
        out_shape=jax.ShapeDtypeStruct((TC_ROWS,), jnp.float32),
        scratch_shapes=[
            pltpu.VMEM((B, D + 1), jnp.float32),
            pltpu.VMEM((B, D + 1), jnp.float32),
        ],
    )(x)


def _epilogue_tc(x_ref, tcmin_ref, pmin_ref, out_ref):
    x = x_ref[...]
    sq = jnp.sum(x * x, axis=1)
    tc = tcmin_ref[...]
    pm = pmin_ref[...]
    d2_sc_part = jnp.min(jnp.min(pm, axis=2), axis=0)
    d2_tc = sq[:TC_ROWS] + tc
    d2_sc = sq[TC_ROWS:] + d2_sc_part
    d2 = jnp.concatenate([d2_tc, d2_sc])
    d2 = jnp.maximum(d2, 0.0)
    nn = jnp.sqrt(d2)
    out_ref[0, 0] = -jnp.mean(jnp.log(nn + 1e-8))


def kernel(student_output):
    x = student_output
    xtp_flat = jnp.transpose(x.T.reshape(D, NW, CW), (1, 0, 2)).reshape(-1)
    pmin = _sc_partial_min(xtp_flat, x.reshape(-1)).reshape(NW, SC_ROWS, L)
    tcmin = _tc_partial_min(x)
    loss = pl.pallas_call(
        _epilogue_tc,
        out_shape=jax.ShapeDtypeStruct((1, 1), jnp.float32),
        out_specs=pl.BlockSpec(memory_space=pltpu.SMEM),
    )(x, tcmin, pmin)
    return loss[0, 0]


# speedup vs baseline: 8.5885x; 1.0444x over previous
import functools

import jax
import jax.numpy as jnp
from jax import lax
from jax.experimental import pallas as pl
from jax.experimental.pallas import tpu as pltpu
from jax.experimental.pallas import tpu_sc as plsc

B = 4096
D = 16
L = 16
NC = 2
NS = 16
NW = NC * NS
NCHUNK = B // L
R = 2

SC_ROWS = 128
TC_ROWS = B - SC_ROWS
SC_ROW0 = TC_ROWS
CW = B // NW
CCH = CW // L
TC_BM = 384
TC_BLOCKS = [(lo, min(lo + TC_BM, TC_ROWS)) for lo in range(0, TC_ROWS, TC_BM)]
assert SC_ROWS % R == 0 and all((hi - lo) % 128 == 0 for lo, hi in TC_BLOCKS)


def _koleo_sc(xtp_hbm, x_hbm, pmin_hbm, xt_v, xrows_v, pmin_v):
    cid = lax.axis_index("c")
    sid = lax.axis_index("s")
    wid = sid * NC + cid
    c0 = wid * CW

    pltpu.sync_copy(xtp_hbm.at[pl.ds(wid * D * CW, D * CW)], xt_v)
    pltpu.sync_copy(x_hbm.at[pl.ds(SC_ROW0 * D, SC_ROWS * D)], xrows_v)

    sjs = []
    for c in range(CCH):
        base = c * L
        ps = []
        for p0 in range(0, D, 4):
            s = None
            for d in range(p0, p0 + 4):
                v = xt_v[pl.ds(d * CW + base, L)]
                s = v * v if s is None else s + v * v
            ps.append(s)
        sjs.append((ps[0] + ps[1]) + (ps[2] + ps[3]))

    inf = jnp.full((L,), jnp.inf, jnp.float32)
    iota = lax.iota(jnp.int32, L)
    dsplat = [jnp.full((L,), d, jnp.int32) for d in range(D)]

    def group(g, carry):
        i0 = SC_ROW0 + g * R
        bs = []
        for r in range(R):
            xi = xrows_v[pl.ds((g * R + r) * D, D)] * -2.0
            for d in range(D):
                bs.append(xi.at[dsplat[d]].get(mode="promise_in_bounds"))
        accs = [inf] * R
        for c in range(CCH):
            base = c * L
            xj = [xt_v[pl.ds(d * CW + base, L)] for d in range(D)]
            jidx = iota + (c0 + base)
            for r in range(R):
                ps = []
                for p0 in range(0, D, 4):
                    m = bs[r * D + p0] * xj[p0]
                    for d in range(p0 + 1, p0 + 4):
                        m = m + bs[r * D + d] * xj[d]
                    ps.append(m)
                m = ((ps[0] + ps[1]) + (ps[2] + ps[3])) + sjs[c]
                diag = jidx == jnp.full((L,), i0 + r, jnp.int32)
                m = jnp.where(diag, inf, m)
                accs[r] = jnp.minimum(accs[r], m)
        for r in range(R):
            pmin_v[pl.ds((g * R + r) * L, L)] = accs[r]
        return carry

    lax.fori_loop(0, SC_ROWS // R, group, 0)
    pltpu.sync_copy(pmin_v, pmin_hbm.at[pl.ds(wid * SC_ROWS * L, SC_ROWS * L)])


def _sc_partial_min(xtp_flat, x_flat):
    mesh = plsc.VectorSubcoreMesh(core_axis_name="c", subcore_axis_name="s")
    kern = pl.kernel(
        _koleo_sc,
        mesh=mesh,
        out_type=jax.ShapeDtypeStruct((NW * SC_ROWS * L,), jnp.float32),
        scratch_types=[
            pltpu.VMEM((D * CW,), jnp.float32),
            pltpu.VMEM((SC_ROWS * D,), jnp.float32),
            pltpu.VMEM((SC_ROWS * L,), jnp.float32),
        ],
    )
    return kern(xtp_flat, x_flat)


def _koleo_tc(x_ref, out_ref, a_ref, b_ref):
    x = x_ref[...]
    sq = jnp.sum(x * x, axis=1, keepdims=True)
    a_ref[...] = jnp.concatenate([x, jnp.ones((B, 1), jnp.float32)], axis=1)
    b_ref[...] = jnp.concatenate([-2.0 * x, sq], axis=1)
    eyes = {}
    for lo, hi in TC_BLOCKS:
        sz = hi - lo
        if sz not in eyes:
            eyes[sz] = (lax.broadcasted_iota(jnp.int32, (sz, sz), 0)
                        == lax.broadcasted_iota(jnp.int32, (sz, sz), 1))
    inf = jnp.float32(jnp.inf)
    bfull = b_ref[...]

    total = None
    for lo, hi in TC_BLOCKS:
        eye = eyes[hi - lo]
        ai = a_ref[lo:hi, :]
        part = lax.dot_general(ai, bfull,
                               (((1,), (1,)), ((), ())),
                               preferred_element_type=jnp.float32)
        strip = jnp.where(eye, inf, part[:, lo:hi])
        m = jnp.min(strip, axis=1)
        if lo > 0:
            m = jnp.minimum(m, jnp.min(part[:, :lo], axis=1))
        if hi < B:
            m = jnp.minimum(m, jnp.min(part[:, hi:], axis=1))
        d2 = jnp.maximum(m + sq[lo:hi, 0], 0.0)
        lg = jnp.sum(jnp.log(jnp.sqrt(d2) + 1e-8))
        total = lg if total is None else total + lg
    out_ref[0, 0] = total


def _tc_partial(x):
    return pl.pallas_call(
        _koleo_tc,
        out_shape=jax.ShapeDtypeStruct((1, 1), jnp.float32),
        out_specs=pl.BlockSpec(memory_space=pltpu.SMEM),
        scratch_shapes=[
            pltpu.VMEM((B, D + 1), jnp.float32),
            pltpu.VMEM((B, D + 1), jnp.float32),
        ],
    )(x)


def _epilogue_tc(xs_ref, tcsum_ref, pmin_ref, out_ref):
    xs = xs_ref[...]
    sq = jnp.sum(xs * xs, axis=1)
    pm = pmin_ref[...]
    d2 = sq + jnp.min(jnp.min(pm, axis=2), axis=0)
    d2 = jnp.maximum(d2, 0.0)
    sc_sum = jnp.sum(jnp.log(jnp.sqrt(d2) + 1e-8))
    out_ref[0, 0] = -(tcsum_ref[0, 0] + sc_sum) / B


def kernel(student_output):
    x = student_output
    xtp_flat = jnp.transpose(x.reshape(NW, CW, D), (0, 2, 1)).reshape(-1)
    pmin = _sc_partial_min(xtp_flat, x.reshape(-1)).reshape(NW, SC_ROWS, L)
    tcsum = _tc_partial(x)
    loss = pl.pallas_call(
        _epilogue_tc,
        out_shape=jax.ShapeDtypeStruct((1, 1), jnp.float32),
        out_specs=pl.BlockSpec(memory_space=pltpu.SMEM),
    )(x[SC_ROW0:], tcsum, pmin)
    return loss[0, 0]


# speedup vs baseline: 9.1194x; 1.0618x over previous
import functools

import jax
import jax.numpy as jnp
from jax import lax
from jax.experimental import pallas as pl
from jax.experimental.pallas import tpu as pltpu
from jax.experimental.pallas import tpu_sc as plsc

B = 4096
D = 16
L = 16
NC = 2
NS = 16
NW = NC * NS
NCHUNK = B // L
R = 2

SC_ROWS = 128
TC_ROWS = B - SC_ROWS
SC_ROW0 = TC_ROWS
CW = B // NW
CCH = CW // L
TC_BM = 384
TC_BLOCKS = [(lo, min(lo + TC_BM, TC_ROWS)) for lo in range(0, TC_ROWS, TC_BM)]
assert SC_ROWS % R == 0 and all((hi - lo) % 128 == 0 for lo, hi in TC_BLOCKS)


def _koleo_sc(xtp_hbm, x_hbm, pmin_hbm, xt_v, xrows_v, pmin_v):
    cid = lax.axis_index("c")
    sid = lax.axis_index("s")
    wid = sid * NC + cid
    c0 = wid * CW

    pltpu.sync_copy(xtp_hbm.at[pl.ds(wid * D * CW, D * CW)], xt_v)
    pltpu.sync_copy(x_hbm.at[pl.ds(SC_ROW0 * D, SC_ROWS * D)], xrows_v)

    sjs = []
    for c in range(CCH):
        base = c * L
        ps = []
        for p0 in range(0, D, 4):
            s = None
            for d in range(p0, p0 + 4):
                v = xt_v[pl.ds(d * CW + base, L)]
                s = v * v if s is None else s + v * v
            ps.append(s)
        sjs.append((ps[0] + ps[1]) + (ps[2] + ps[3]))

    inf = jnp.full((L,), jnp.inf, jnp.float32)
    iota = lax.iota(jnp.int32, L)
    dsplat = [jnp.full((L,), d, jnp.int32) for d in range(D)]

    def group(g, carry):
        i0 = SC_ROW0 + g * R
        bs = []
        for r in range(R):
            xi = xrows_v[pl.ds((g * R + r) * D, D)] * -2.0
            for d in range(D):
                bs.append(xi.at[dsplat[d]].get(mode="promise_in_bounds"))
        accs = [inf] * R
        for c in range(CCH):
            base = c * L
            xj = [xt_v[pl.ds(d * CW + base, L)] for d in range(D)]
            jidx = iota + (c0 + base)
            for r in range(R):
                ps = []
                for p0 in range(0, D, 4):
                    m = bs[r * D + p0] * xj[p0]
                    for d in range(p0 + 1, p0 + 4):
                        m = m + bs[r * D + d] * xj[d]
                    ps.append(m)
                m = ((ps[0] + ps[1]) + (ps[2] + ps[3])) + sjs[c]
                diag = jidx == jnp.full((L,), i0 + r, jnp.int32)
                m = jnp.where(diag, inf, m)
                accs[r] = jnp.minimum(accs[r], m)
        for r in range(R):
            pmin_v[g * R + r, :] = accs[r]
        return carry

    lax.fori_loop(0, SC_ROWS // R, group, 0)
    pltpu.sync_copy(pmin_v, pmin_hbm.at[pl.ds(wid * SC_ROWS, SC_ROWS), :])


def _sc_partial_min(xtp_flat, x_flat):
    mesh = plsc.VectorSubcoreMesh(core_axis_name="c", subcore_axis_name="s")
    kern = pl.kernel(
        _koleo_sc,
        mesh=mesh,
        out_type=jax.ShapeDtypeStruct((NW * SC_ROWS, L), jnp.float32),
        scratch_types=[
            pltpu.VMEM((D * CW,), jnp.float32),
            pltpu.VMEM((SC_ROWS * D,), jnp.float32),
            pltpu.VMEM((SC_ROWS, L), jnp.float32),
        ],
    )
    return kern(xtp_flat, x_flat)


def _koleo_tc(x_ref, out_ref, a_ref, b_ref):
    x = x_ref[...]
    sq = jnp.sum(x * x, axis=1, keepdims=True)
    a_ref[...] = jnp.concatenate([x, jnp.ones((B, 1), jnp.float32)], axis=1)
    b_ref[...] = jnp.concatenate([-2.0 * x, sq], axis=1)
    eyes = {}
    for lo, hi in TC_BLOCKS:
        sz = hi - lo
        if sz not in eyes:
            eyes[sz] = (lax.broadcasted_iota(jnp.int32, (sz, sz), 0)
                        == lax.broadcasted_iota(jnp.int32, (sz, sz), 1))
    inf = jnp.float32(jnp.inf)
    bfull = b_ref[...]

    total = None
    for lo, hi in TC_BLOCKS:
        eye = eyes[hi - lo]
        ai = a_ref[lo:hi, :]
        part = lax.dot_general(ai, bfull,
                               (((1,), (1,)), ((), ())),
                               preferred_element_type=jnp.float32)
        strip = jnp.where(eye, inf, part[:, lo:hi])
        m = jnp.min(strip, axis=1)
        if lo > 0:
            m = jnp.minimum(m, jnp.min(part[:, :lo], axis=1))
        if hi < B:
            m = jnp.minimum(m, jnp.min(part[:, hi:], axis=1))
        d2 = jnp.maximum(m + sq[lo:hi, 0], 0.0)
        lg = jnp.sum(jnp.log(jnp.sqrt(d2) + 1e-8))
        total = lg if total is None else total + lg
    out_ref[0, 0] = total


def _tc_partial(x):
    return pl.pallas_call(
        _koleo_tc,
        out_shape=jax.ShapeDtypeStruct((1, 1), jnp.float32),
        out_specs=pl.BlockSpec(memory_space=pltpu.SMEM),
        scratch_shapes=[
            pltpu.VMEM((B, D + 1), jnp.float32),
            pltpu.VMEM((B, D + 1), jnp.float32),
        ],
    )(x)


def _epilogue_tc(x_ref, tcsum_ref, pmin_ref, out_ref):
    xs = x_ref[SC_ROW0:, :]
    sq = jnp.sum(xs * xs, axis=1)
    pv = jnp.min(pmin_ref[...], axis=1)
    m = pv[:SC_ROWS]
    for w in range(1, NW):
        m = jnp.minimum(m, pv[w * SC_ROWS:(w + 1) * SC_ROWS])
    d2 = sq + m
    d2 = jnp.maximum(d2, 0.0)
    sc_sum = jnp.sum(jnp.log(jnp.sqrt(d2) + 1e-8))
    out_ref[0, 0] = -(tcsum_ref[0, 0] + sc_sum) / B


def kernel(student_output):
    x = student_output
    xtp_flat = jnp.transpose(x.reshape(NW, CW, D), (0, 2, 1)).reshape(-1)
    pmin = _sc_partial_min(xtp_flat, x.reshape(-1))
    tcsum = _tc_partial(x)
    loss = pl.pallas_call(
        _epilogue_tc,
        out_shape=jax.ShapeDtypeStruct((1, 1), jnp.float32),
        out_specs=pl.BlockSpec(memory_space=pltpu.SMEM),
    )(x, tcsum, pmin)
    return loss[0, 0]


# speedup vs baseline: 9.9369x; 1.0896x over previous
import functools

import jax
import jax.numpy as jnp
from jax import lax
from jax.experimental import pallas as pl
from jax.experimental.pallas import tpu as pltpu
from jax.experimental.pallas import tpu_sc as plsc

B = 4096
D = 16
L = 16
NC = 2
NS = 16
NW = NC * NS
NCHUNK = B // L
R = 2

SC_ROWS = 128
TC_ROWS = B - SC_ROWS
SC_ROW0 = TC_ROWS
CW = B // NW
CCH = CW // L
TC_BM = 384
TC_BLOCKS = [(lo, min(lo + TC_BM, TC_ROWS)) for lo in range(0, TC_ROWS, TC_BM)]
assert all((hi - lo) % 128 == 0 for lo, hi in TC_BLOCKS)


RPG = 8
NKW = NW // 2
SCCH = SC_ROWS // L
assert NKW * RPG == SC_ROWS


def _koleo_sc(xts_hbm, x_hbm, pmin_hbm, xt_v, xrows_v, pmin_v):
    cid = lax.axis_index("c")
    sid = lax.axis_index("s")
    wid = sid * NC + cid
    k = wid // 2
    par = wid % 2

    pltpu.sync_copy(xts_hbm, xt_v)
    pltpu.sync_copy(x_hbm.at[pl.ds(SC_ROW0 + k * RPG, RPG), :], xrows_v)

    NCH = SCCH // 2
    cb0 = par * (NCH * L)
    sjs = []
    for c in range(NCH):
        base = cb0 + c * L
        ps = []
        for p0 in range(0, D, 4):
            s = None
            for d in range(p0, p0 + 4):
                v = xt_v[d, pl.ds(base, L)]
                s = v * v if s is None else s + v * v
            ps.append(s)
        sjs.append((ps[0] + ps[1]) + (ps[2] + ps[3]))

    inf = jnp.full((L,), jnp.inf, jnp.float32)
    iota = lax.iota(jnp.int32, L)
    dsplat = [jnp.full((L,), d, jnp.int32) for d in range(D)]

    for g in range(RPG // R):
        row0 = g * R
        i0 = SC_ROW0 + k * RPG + row0
        bs = []
        for r in range(R):
            xi = xrows_v[row0 + r, :] * -2.0
            for d in range(D):
                bs.append(xi.at[dsplat[d]].get(mode="promise_in_bounds"))
        accs = [inf] * R
        for c in range(NCH):
            base = cb0 + c * L
            xj = [xt_v[d, pl.ds(base, L)] for d in range(D)]
            jidx = iota + (SC_ROW0 + base)
            for r in range(R):
                ps = []
                for p0 in range(0, D, 4):
                    m = bs[r * D + p0] * xj[p0]
                    for d in range(p0 + 1, p0 + 4):
                        m = m + bs[r * D + d] * xj[d]
                    ps.append(m)
                m = ((ps[0] + ps[1]) + (ps[2] + ps[3])) + sjs[c]
                diag = jidx == jnp.full((L,), i0 + r, jnp.int32)
                m = jnp.where(diag, inf, m)
                accs[r] = jnp.minimum(accs[r], m)
        for r in range(R):
            pmin_v[row0 + r, :] = accs[r]

    pltpu.sync_copy(pmin_v, pmin_hbm.at[par, pl.ds(k * RPG, RPG), :])


def _sc_partial_min(xts, x):
    mesh = plsc.VectorSubcoreMesh(core_axis_name="c", subcore_axis_name="s")
    kern = pl.kernel(
        _koleo_sc,
        mesh=mesh,
        out_type=jax.ShapeDtypeStruct((2, SC_ROWS, L), jnp.float32),
        scratch_types=[
            pltpu.VMEM((D, SC_ROWS), jnp.float32),
            pltpu.VMEM((RPG, D), jnp.float32),
            pltpu.VMEM((RPG, L), jnp.float32),
        ],
    )
    return kern(xts, x)


def _koleo_tc(x_ref, out_ref, scmin_ref, a_ref, b_ref):
    x = x_ref[...]
    sq = jnp.sum(x * x, axis=1, keepdims=True)
    ones = jnp.ones((B, 1), jnp.float32)
    a_ref[...] = jnp.concatenate([x, ones, sq], axis=1)
    b_ref[...] = jnp.concatenate([-2.0 * x, sq, ones], axis=1)
    eyes = {}
    for lo, hi in TC_BLOCKS:
        sz = hi - lo
        if sz not in eyes:
            eyes[sz] = (lax.broadcasted_iota(jnp.int32, (sz, sz), 0)
                        == lax.broadcasted_iota(jnp.int32, (sz, sz), 1))
    inf = jnp.float32(jnp.inf)

    colacc = jnp.full((TC_ROWS,), jnp.inf, jnp.float32)
    scacc = jnp.full((SC_ROWS,), jnp.inf, jnp.float32)
    total = None
    for lo, hi in TC_BLOCKS:
        sz = hi - lo
        eye = eyes[sz]
        ai = a_ref[lo:hi, :]
        bj = b_ref[lo:, :]
        part = lax.dot_general(ai, bj,
                               (((1,), (1,)), ((), ())),
                               preferred_element_type=jnp.float32)
        strip = jnp.where(eye, inf, part[:, :sz])
        m = jnp.min(strip, axis=1)
        if hi < B:
            m = jnp.minimum(m, jnp.min(part[:, sz:], axis=1))
        m = jnp.minimum(m, colacc[lo:hi])
        if hi < TC_ROWS:
            cmin = jnp.min(part[:, sz:TC_ROWS - lo], axis=0)
            colacc = jnp.concatenate(
                [colacc[:hi], jnp.minimum(colacc[hi:], cmin)])
        scacc = jnp.minimum(scacc, jnp.min(part[:, TC_ROWS - lo:], axis=0))
        d2 = jnp.maximum(m, 0.0)
        lg = jnp.sum(jnp.log(jnp.sqrt(d2) + 1e-8))
        total = lg if total is None else total + lg
    out_ref[0, 0] = total
    scmin_ref[...] = scacc


def _tc_partial(x):
    return pl.pallas_call(
        _koleo_tc,
        out_shape=[
            jax.ShapeDtypeStruct((1, 1), jnp.float32),
            jax.ShapeDtypeStruct((SC_ROWS,), jnp.float32),
        ],
        out_specs=[
            pl.BlockSpec(memory_space=pltpu.SMEM),
            pl.BlockSpec(memory_space=pltpu.VMEM),
        ],
        scratch_shapes=[
            pltpu.VMEM((B, D + 2), jnp.float32),
            pltpu.VMEM((B, D + 2), jnp.float32),
        ],
    )(x)


def _epilogue_tc(x_ref, tcsum_ref, scmin_ref, pmin_ref, out_ref):
    xs = x_ref[SC_ROW0:, :]
    sq = jnp.sum(xs * xs, axis=1)
    pm = pmin_ref[...]
    own = sq + jnp.min(jnp.min(pm, axis=0), axis=1)
    d2 = jnp.minimum(own, scmin_ref[...])
    d2 = jnp.maximum(d2, 0.0)
    sc_sum = jnp.sum(jnp.log(jnp.sqrt(d2) + 1e-8))
    out_ref[0, 0] = -(tcsum_ref[0, 0] + sc_sum) / B


def kernel(student_output):
    x = student_output
    xts = x[SC_ROW0:].T
    pmin = _sc_partial_min(xts, x)
    tcsum, scmin = _tc_partial(x)
    loss = pl.pallas_call(
        _epilogue_tc,
        out_shape=jax.ShapeDtypeStruct((1, 1), jnp.float32),
        out_specs=pl.BlockSpec(memory_space=pltpu.SMEM),
    )(x, tcsum, scmin, pmin)
    return loss[0, 0]
